# Initial kernel scaffold; baseline (speedup 1.0000x reference)
import jax
import jax.numpy as jnp
from jax.experimental import pallas as pl


def kernel(actions, action_table):
    raise NotImplementedError("write your pallas kernel here")


# speedup vs baseline: 1.7500x; 1.7500x over previous
import functools
import jax
import jax.numpy as jnp
from jax import lax
from jax.experimental import pallas as pl
from jax.experimental.pallas import tpu as pltpu
from jax.experimental.pallas import tpu_sc as plsc

D = 128
BOS = 98
L = 4
S = L + 1


def kernel(actions, action_table):
    B = actions.shape[0]
    NC, NS = 2, 16
    NW = NC * NS
    b_per_w = B // NW
    C = 64
    n_sub = b_per_w // C

    actions_t = actions.T.reshape(L * B)
    mesh = plsc.VectorSubcoreMesh(core_axis_name="c", subcore_axis_name="s")

    @functools.partial(
        pl.kernel,
        out_type=jax.ShapeDtypeStruct((B, S, D), jnp.float32),
        mesh=mesh,
        scratch_types=[
            pltpu.VMEM((L, C), jnp.int32),
            pltpu.VMEM((C, S, D), jnp.float32),
            pltpu.SemaphoreType.DMA,
        ],
    )
    def emb_kernel(actions_hbm, table_hbm, out_hbm, idx_v, big_v, sem):
        wid = lax.axis_index("s") * NC + lax.axis_index("c")

        pltpu.sync_copy(table_hbm.at[pl.ds(BOS, 1)], big_v.at[0, pl.ds(0, 1)])

        def rep(i, carry):
            for j in range(D // 16):
                big_v[i, 0, pl.ds(j * 16, 16)] = big_v[0, 0, pl.ds(j * 16, 16)]
            return carry

        lax.fori_loop(1, C, rep, 0)

        def sub(s, carry):
            base = wid * b_per_w + s * C
            for l in range(L):
                pltpu.sync_copy(actions_hbm.at[pl.ds(l * B + base, C)], idx_v.at[l])
            copies = [
                pltpu.async_copy(table_hbm.at[idx_v.at[l]], big_v.at[:, 1 + l], sem)
                for l in range(L)
            ]
            for cp in copies:
                cp.wait()
            pltpu.sync_copy(big_v, out_hbm.at[pl.ds(base, C)])
            return carry

        lax.fori_loop(0, n_sub, sub, 0)

    return emb_kernel(actions_t, action_table)
